# Initial kernel scaffold; baseline (speedup 1.0000x reference)
import jax
import jax.numpy as jnp
from jax.experimental import pallas as pl


def kernel(features, W0, b0, W1, b1):
    raise NotImplementedError("write your pallas kernel here")


# speedup vs baseline: 14.7477x; 14.7477x over previous
import functools

import jax
import jax.numpy as jnp
from jax.experimental import pallas as pl

_KP1 = 31
_BISECT_ITERS = 30


def _mlp_kernel(f_ref, w0_ref, b0_ref, w1_ref, b1_ref, emb_ref):
    f = f_ref[...]
    h = jax.lax.dot_general(
        f, w0_ref[...], (((1,), (1,)), ((), ())),
        preferred_element_type=jnp.float32)
    h = jnp.maximum(h + b0_ref[...], 0.0)
    h = jax.lax.dot_general(
        h, w1_ref[...], (((1,), (1,)), ((), ())),
        preferred_element_type=jnp.float32)
    h = h + b1_ref[...]
    norm = jnp.sqrt(jnp.sum(h * h, axis=1, keepdims=True))
    norm = jnp.maximum(norm, 1e-12)
    emb_ref[...] = h / norm


def _sim_topk_kernel(rows_ref, embt_ref, out_ref, *, block_m):
    s = jax.lax.dot_general(
        rows_ref[...], embt_ref[...], (((1,), (0,)), ((), ())),
        preferred_element_type=jnp.float32)
    out_ref[...] = s

    def body(_, carry):
        lo, hi = carry
        mid = 0.5 * (lo + hi)
        cnt = jnp.sum((out_ref[...] >= mid).astype(jnp.float32), axis=1,
                      keepdims=True)
        ge = cnt >= float(_KP1)
        return jnp.where(ge, mid, lo), jnp.where(ge, hi, mid)

    lo0 = jnp.zeros((block_m, 1), jnp.float32)
    hi0 = jnp.full((block_m, 1), 2.0, jnp.float32)
    lo, _ = jax.lax.fori_loop(0, _BISECT_ITERS, body, (lo0, hi0))
    s = out_ref[...]
    out_ref[...] = jnp.where(s >= lo, s, 0.0)


def _pick_block_m(n):
    for m in (400, 256, 200, 128, 80, 64, 40, 32, 16, 8):
        if n % m == 0:
            return m
    return n


def kernel(features, W0, b0, W1, b1):
    n, d = features.shape
    emb = pl.pallas_call(
        _mlp_kernel,
        out_shape=jax.ShapeDtypeStruct((n, d), jnp.float32),
    )(features, W0, b0.reshape(1, d), W1, b1.reshape(1, d))

    embt = emb.T
    block_m = _pick_block_m(n)
    grid = n // block_m
    out = pl.pallas_call(
        functools.partial(_sim_topk_kernel, block_m=block_m),
        grid=(grid,),
        in_specs=[
            pl.BlockSpec((block_m, d), lambda i: (i, 0)),
            pl.BlockSpec((d, n), lambda i: (0, 0)),
        ],
        out_specs=pl.BlockSpec((block_m, n), lambda i: (i, 0)),
        out_shape=jax.ShapeDtypeStruct((n, n), jnp.float32),
    )(emb, embt)
    return out
